# baseline (speedup 1.0000x reference)
import functools

import jax
import jax.numpy as jnp
from jax import lax
from jax.experimental import pallas as pl
from jax.experimental.pallas import tpu as pltpu
from jax.experimental.pallas import tpu_sc as plsc

_DIM = 2048
_HALF = _DIM // 2
_BS = 512


def _sc_gather(label_table, labels):
    batch = labels.shape[0]
    mesh = plsc.VectorSubcoreMesh(core_axis_name="c", subcore_axis_name="s")

    @functools.partial(
        pl.kernel,
        mesh=mesh,
        out_type=jax.ShapeDtypeStruct((batch, _DIM), jnp.float32),
        scratch_types=[
            pltpu.VMEM((batch,), jnp.int32),
            pltpu.VMEM((batch, _DIM), jnp.float32),
            pltpu.SemaphoreType.DMA,
        ],
    )
    def gather_kernel(table_hbm, idx_hbm, out_hbm, idx_v, rows_v, sem):
        wid = lax.axis_index("s") * 2 + lax.axis_index("c")

        @pl.when(wid == 0)
        def _():
            pltpu.sync_copy(idx_hbm, idx_v)
            pltpu.async_copy(table_hbm.at[idx_v], rows_v, sem).wait()
            pltpu.sync_copy(rows_v, out_hbm)

    return gather_kernel(label_table, labels)


def _inv_freq(shape):
    d = jax.lax.broadcasted_iota(jnp.int32, shape, 1).astype(jnp.float32)
    return jnp.exp(d * (-jnp.log(10000.0) / _HALF))


def _rope_kernel(x_ref, le_ref, o_ref, emb_ref):
    s_blk = pl.program_id(0)
    b = pl.program_id(1)

    @pl.when(jnp.logical_and(s_blk == 0, b == 0))
    def _init_block0():
        k = jax.lax.broadcasted_iota(jnp.int32, (_BS, _HALF), 0).astype(jnp.float32)
        ang = k * _inv_freq((_BS, _HALF))
        emb_ref[:, :_HALF] = jnp.sin(ang)
        emb_ref[:, _HALF:] = jnp.cos(ang)

    @pl.when(jnp.logical_and(s_blk > 0, b == 0))
    def _advance_block():
        ang_d = jnp.float32(_BS) * _inv_freq((1, _HALF))
        sin_d = jnp.sin(ang_d)
        cos_d = jnp.cos(ang_d)
        es = emb_ref[:, :_HALF]
        ec = emb_ref[:, _HALF:]
        emb_ref[:, :_HALF] = es * cos_d + ec * sin_d
        emb_ref[:, _HALF:] = ec * cos_d - es * sin_d

    le = le_ref[0, 0, :]
    o_ref[0] = x_ref[0] + emb_ref[...] * le[None, :]


def kernel(x, labels, label_table):
    batch, seq, dim = x.shape
    assert dim == _DIM and seq % _BS == 0
    labels = labels.astype(jnp.int32)

    label_embeds = _sc_gather(label_table, labels)
    le3 = label_embeds.reshape(batch, 1, dim)

    grid = (seq // _BS, batch)
    return pl.pallas_call(
        _rope_kernel,
        grid=grid,
        in_specs=[
            pl.BlockSpec((1, _BS, _DIM), lambda s, b: (b, s, 0)),
            pl.BlockSpec((1, 1, _DIM), lambda s, b: (b, 0, 0)),
        ],
        out_specs=pl.BlockSpec((1, _BS, _DIM), lambda s, b: (b, s, 0)),
        scratch_shapes=[
            pltpu.VMEM((_BS, _DIM), jnp.float32),
        ],
        out_shape=jax.ShapeDtypeStruct(x.shape, x.dtype),
        compiler_params=pltpu.CompilerParams(
            dimension_semantics=("arbitrary", "arbitrary"),
        ),
    )(x, le3)


# speedup vs baseline: 1.1130x; 1.1130x over previous
import jax
import jax.numpy as jnp
from jax.experimental import pallas as pl
from jax.experimental.pallas import tpu as pltpu

_DIM = 2048
_HALF = _DIM // 2
_BS = 512


def _inv_freq(shape):
    d = jax.lax.broadcasted_iota(jnp.int32, shape, 1).astype(jnp.float32)
    return jnp.exp(d * (-jnp.log(10000.0) / _HALF))


def _rope_kernel(labels_ref, x_ref, table_ref, o_ref, emb_ref):
    del labels_ref
    s_blk = pl.program_id(0)
    b = pl.program_id(1)

    @pl.when(jnp.logical_and(s_blk == 0, b == 0))
    def _init_block0():
        k = jax.lax.broadcasted_iota(jnp.int32, (_BS, _HALF), 0).astype(jnp.float32)
        ang = k * _inv_freq((_BS, _HALF))
        emb_ref[:, :_HALF] = jnp.sin(ang)
        emb_ref[:, _HALF:] = jnp.cos(ang)

    @pl.when(jnp.logical_and(s_blk > 0, b == 0))
    def _advance_block():
        ang_d = jnp.float32(_BS) * _inv_freq((1, _HALF))
        sin_d = jnp.sin(ang_d)
        cos_d = jnp.cos(ang_d)
        es = emb_ref[:, :_HALF]
        ec = emb_ref[:, _HALF:]
        emb_ref[:, :_HALF] = es * cos_d + ec * sin_d
        emb_ref[:, _HALF:] = ec * cos_d - es * sin_d

    le = table_ref[0, 0, :]
    o_ref[0] = x_ref[0] + emb_ref[...] * le[None, :]


def kernel(x, labels, label_table):
    batch, seq, dim = x.shape
    assert dim == _DIM and seq % _BS == 0
    labels = labels.astype(jnp.int32)
    table3 = label_table.reshape(label_table.shape[0], 1, dim)
    grid = (seq // _BS, batch)
    return pl.pallas_call(
        _rope_kernel,
        grid_spec=pltpu.PrefetchScalarGridSpec(
            num_scalar_prefetch=1,
            grid=grid,
            in_specs=[
                pl.BlockSpec((1, _BS, _DIM), lambda s, b, labels: (b, s, 0)),
                pl.BlockSpec((1, 1, _DIM), lambda s, b, labels: (labels[b], 0, 0)),
            ],
            out_specs=pl.BlockSpec((1, _BS, _DIM), lambda s, b, labels: (b, s, 0)),
            scratch_shapes=[
                pltpu.VMEM((_BS, _DIM), jnp.float32),
            ],
        ),
        out_shape=jax.ShapeDtypeStruct(x.shape, x.dtype),
        compiler_params=pltpu.CompilerParams(
            dimension_semantics=("arbitrary", "arbitrary"),
        ),
    )(labels, x, table3)
